# baseline (speedup 1.0000x reference)
import numpy as np
import jax
import jax.numpy as jnp
from jax import lax
from jax.experimental import pallas as pl
from jax.experimental.pallas import tpu as pltpu
from jax.experimental.pallas import tpu_sc as plsc

_F = 26
_V = 100000
_D = 16
_B = 4096
_NC, _NS, _L = 2, 16, 16
_NW = _NC * _NS
_BPW = _B // _NW
_P = (_F * (_F - 1)) // 2
_XP = 32
_W = 416
_VC = 2048


def _tbody(x_ref, o_ref):
    x = x_ref[...].reshape(_F * _D, _VC)
    o_ref[...] = jnp.transpose(x, (1, 0))


def _transpose_tables(tables):
    tview = jnp.transpose(tables, (0, 2, 1))
    return pl.pallas_call(
        _tbody,
        grid=((_V + _VC - 1) // _VC,),
        in_specs=[pl.BlockSpec((_F, _D, _VC), lambda vc: (0, 0, vc))],
        out_specs=pl.BlockSpec((_VC, _W), lambda vc: (vc, 0)),
        out_shape=jax.ShapeDtypeStruct((_V, _W), jnp.float32),
    )(tview)


def _body(z_hbm, xp_hbm, out_hbm, x_v, rows_a, rows_b, out2_v,
          sem_a, sem_b):
    wid = lax.axis_index("s") * _NC + lax.axis_index("c")
    lanes = jnp.arange(_L, dtype=jnp.int32)
    zeros = jnp.zeros((_L,), jnp.float32)

    def fire(n, rows_ref, sem):
        ix = x_v.at[pl.ds(n * _XP, _F)]
        pltpu.async_copy(z_hbm.at[ix], rows_ref, sem)

    def drain(rows_ref, sem):
        pltpu.make_async_copy(z_hbm.at[pl.ds(0, _F)], rows_ref, sem).wait()

    def compute(rows_ref):
        accs = [jnp.zeros((_L,), jnp.float32) for _ in range(8)]
        k = 0
        for i in range(_F):
            for j in range(i + 1, _F):
                accs[k % 8] = accs[k % 8] + (
                    rows_ref[j, pl.ds(i * _D, _D)]
                    * rows_ref[i, pl.ds(j * _D, _D)])
                k += 1
        acc = ((accs[0] + accs[1]) + (accs[2] + accs[3])) + (
            (accs[4] + accs[5]) + (accs[6] + accs[7]))
        return jnp.sum(acc)

    pltpu.sync_copy(xp_hbm.at[pl.ds(wid * _BPW * _XP, _BPW * _XP)], x_v)
    fire(0, rows_a, sem_a)
    fire(1, rows_b, sem_b)

    def step(k, r):
        def half(b, nxt, rows_ref, sem, r):
            drain(rows_ref, sem)
            s = compute(rows_ref)
            lane = jnp.bitwise_and(b, _L - 1)
            r = jnp.where(lanes == lane, s, r)
            out2_v[lax.shift_right_logical(b, 4)] = r

            @pl.when(k < _BPW // 2 - 1)
            def _():
                fire(nxt, rows_ref, sem)

            return jnp.where(lane == _L - 1, zeros, r)

        b0 = 2 * k
        r = half(b0, b0 + 2, rows_a, sem_a, r)
        r = half(b0 + 1, b0 + 3, rows_b, sem_b, r)
        return r

    lax.fori_loop(0, _BPW // 2, step, zeros)
    pltpu.sync_copy(out2_v, out_hbm.at[wid])


def kernel(x, tables):
    z = _transpose_tables(tables)
    xp = jnp.pad(x.astype(jnp.int32), ((0, 0), (0, _XP - _F))).reshape(-1)
    mesh = plsc.VectorSubcoreMesh(core_axis_name="c", subcore_axis_name="s",
                                  num_cores=_NC, num_subcores=_NS)
    out = pl.kernel(
        _body,
        out_type=jax.ShapeDtypeStruct((_NW, _BPW // _L, _L), jnp.float32),
        mesh=mesh,
        compiler_params=pltpu.CompilerParams(needs_layout_passes=False,
                                             use_tc_tiling_on_sc=False),
        scratch_types=[
            pltpu.VMEM((_BPW * _XP,), jnp.int32),
            pltpu.VMEM((_F, _W), jnp.float32),
            pltpu.VMEM((_F, _W), jnp.float32),
            pltpu.VMEM((_BPW // _L, _L), jnp.float32),
            pltpu.SemaphoreType.DMA,
            pltpu.SemaphoreType.DMA,
        ],
    )(z, xp)
    return out.reshape(_B, 1)


# speedup vs baseline: 1.0711x; 1.0711x over previous
import numpy as np
import jax
import jax.numpy as jnp
from jax import lax
from jax.experimental import pallas as pl
from jax.experimental.pallas import tpu as pltpu
from jax.experimental.pallas import tpu_sc as plsc

_F = 26
_V = 100000
_D = 16
_B = 4096
_NC, _NS, _L = 2, 16, 16
_NW = _NC * _NS
_BPW = _B // _NW
_P = (_F * (_F - 1)) // 2
_XP = 32
_W = 512
_VC = 2048


def _tbody(x_ref, o_ref):
    x = x_ref[...].reshape(_F * _D, _VC)
    o_ref[:, 0:_F * _D] = jnp.transpose(x, (1, 0))


def _transpose_tables(tables):
    tview = jnp.transpose(tables, (0, 2, 1))
    return pl.pallas_call(
        _tbody,
        grid=((_V + _VC - 1) // _VC,),
        in_specs=[pl.BlockSpec((_F, _D, _VC), lambda vc: (0, 0, vc))],
        out_specs=pl.BlockSpec((_VC, _W), lambda vc: (vc, 0)),
        out_shape=jax.ShapeDtypeStruct((_V, _W), jnp.float32),
    )(tview)


def _body(z_hbm, xp_hbm, out_hbm, x_v, rows_a, rows_b, rows_c, rows_d,
          out2_v, sem_a, sem_b, sem_c, sem_d):
    wid = lax.axis_index("s") * _NC + lax.axis_index("c")
    lanes = jnp.arange(_L, dtype=jnp.int32)
    zeros = jnp.zeros((_L,), jnp.float32)

    def fire(n, rows_ref, sem):
        ix = x_v.at[pl.ds(n * _XP, _F)]
        pltpu.async_copy(z_hbm.at[ix], rows_ref, sem)

    def drain(rows_ref, sem):
        pltpu.make_async_copy(z_hbm.at[pl.ds(0, _F)], rows_ref, sem).wait()

    def compute2(ra, rb):
        accs = [jnp.zeros((_L,), jnp.float32) for _ in range(8)]
        k = 0
        for i in range(_F):
            for j in range(i + 1, _F):
                accs[k % 4] = accs[k % 4] + (
                    ra[j, pl.ds(i * _D, _D)] * ra[i, pl.ds(j * _D, _D)])
                accs[4 + k % 4] = accs[4 + k % 4] + (
                    rb[j, pl.ds(i * _D, _D)] * rb[i, pl.ds(j * _D, _D)])
                k += 1
        sa = jnp.sum((accs[0] + accs[1]) + (accs[2] + accs[3]))
        sb = jnp.sum((accs[4] + accs[5]) + (accs[6] + accs[7]))
        return sa, sb

    pltpu.sync_copy(xp_hbm.at[pl.ds(wid * _BPW * _XP, _BPW * _XP)], x_v)
    fire(0, rows_a, sem_a)
    fire(1, rows_b, sem_b)
    fire(2, rows_c, sem_c)
    fire(3, rows_d, sem_d)

    def step(k, r):
        b0 = 4 * k
        l0 = jnp.bitwise_and(b0, _L - 1)

        def phase(off, r0, s0, r1, s1, r):
            drain(r0, s0)
            drain(r1, s1)
            sa, sb = compute2(r0, r1)
            r = jnp.where(lanes == l0 + off, sa, r)
            r = jnp.where(lanes == l0 + off + 1, sb, r)

            @pl.when(k < _BPW // 4 - 1)
            def _():
                fire(b0 + off + 4, r0, s0)
                fire(b0 + off + 5, r1, s1)

            return r

        r = phase(0, rows_a, sem_a, rows_b, sem_b, r)
        r = phase(2, rows_c, sem_c, rows_d, sem_d, r)
        out2_v[lax.shift_right_logical(b0, 4)] = r
        return jnp.where(l0 == _L - 4, zeros, r)

    lax.fori_loop(0, _BPW // 4, step, zeros)
    pltpu.sync_copy(out2_v, out_hbm.at[wid])


def kernel(x, tables):
    z = _transpose_tables(tables)
    xp = jnp.pad(x.astype(jnp.int32), ((0, 0), (0, _XP - _F))).reshape(-1)
    mesh = plsc.VectorSubcoreMesh(core_axis_name="c", subcore_axis_name="s",
                                  num_cores=_NC, num_subcores=_NS)
    out = pl.kernel(
        _body,
        out_type=jax.ShapeDtypeStruct((_NW, _BPW // _L, _L), jnp.float32),
        mesh=mesh,
        compiler_params=pltpu.CompilerParams(needs_layout_passes=False,
                                             use_tc_tiling_on_sc=False),
        scratch_types=[
            pltpu.VMEM((_BPW * _XP,), jnp.int32),
            pltpu.VMEM((_F, _W), jnp.float32),
            pltpu.VMEM((_F, _W), jnp.float32),
            pltpu.VMEM((_F, _W), jnp.float32),
            pltpu.VMEM((_F, _W), jnp.float32),
            pltpu.VMEM((_BPW // _L, _L), jnp.float32),
            pltpu.SemaphoreType.DMA,
            pltpu.SemaphoreType.DMA,
            pltpu.SemaphoreType.DMA,
            pltpu.SemaphoreType.DMA,
        ],
    )(z, xp)
    return out.reshape(_B, 1)


# speedup vs baseline: 1.1481x; 1.0719x over previous
import numpy as np
import jax
import jax.numpy as jnp
from jax import lax
from jax.experimental import pallas as pl
from jax.experimental.pallas import tpu as pltpu
from jax.experimental.pallas import tpu_sc as plsc

_F = 26
_V = 100000
_D = 16
_B = 4096
_NC, _NS, _L = 2, 16, 16
_NW = _NC * _NS
_BPW = _B // _NW
_P = (_F * (_F - 1)) // 2
_XP = 32
_W = 512
_VC = 2048


def _tbody(x_ref, o_ref):
    x = x_ref[...].reshape(_F * _D, _VC)
    o_ref[:, 0:_F * _D] = jnp.transpose(x, (1, 0))


def _transpose_tables(tables):
    tview = jnp.transpose(tables, (0, 2, 1))
    return pl.pallas_call(
        _tbody,
        grid=((_V + _VC - 1) // _VC,),
        in_specs=[pl.BlockSpec((_F, _D, _VC), lambda vc: (0, 0, vc))],
        out_specs=pl.BlockSpec((_VC, _W), lambda vc: (vc, 0)),
        out_shape=jax.ShapeDtypeStruct((_V, _W), jnp.float32),
    )(tview)


def _body(z_hbm, xp_hbm, out_hbm, x_v, rows_a, rows_b, out2_v,
          sem_a, sem_b):
    wid = lax.axis_index("s") * _NC + lax.axis_index("c")
    lanes = jnp.arange(_L, dtype=jnp.int32)
    zeros = jnp.zeros((_L,), jnp.float32)

    def fire(n, rows_ref, sem):
        ix = x_v.at[pl.ds(n * _XP, _F)]
        pltpu.async_copy(z_hbm.at[ix], rows_ref, sem)

    def drain(rows_ref, sem):
        pltpu.make_async_copy(z_hbm.at[pl.ds(0, _F)], rows_ref, sem).wait()

    def compute(rows_ref):
        accs = [jnp.zeros((_L,), jnp.float32) for _ in range(8)]
        k = 0
        for i in range(_F):
            for j in range(i + 1, _F):
                accs[k % 8] = accs[k % 8] + (
                    rows_ref[j, pl.ds(i * _D, _D)]
                    * rows_ref[i, pl.ds(j * _D, _D)])
                k += 1
        acc = ((accs[0] + accs[1]) + (accs[2] + accs[3])) + (
            (accs[4] + accs[5]) + (accs[6] + accs[7]))
        return jnp.sum(acc)

    pltpu.sync_copy(xp_hbm.at[pl.ds(wid * _BPW * _XP, _BPW * _XP)], x_v)
    fire(0, rows_a, sem_a)
    fire(1, rows_b, sem_b)

    def step(k, r):
        def half(b, nxt, rows_ref, sem, r):
            drain(rows_ref, sem)
            s = compute(rows_ref)
            lane = jnp.bitwise_and(b, _L - 1)
            r = jnp.where(lanes == lane, s, r)
            out2_v[lax.shift_right_logical(b, 4)] = r

            @pl.when(k < _BPW // 2 - 1)
            def _():
                fire(nxt, rows_ref, sem)

            return jnp.where(lane == _L - 1, zeros, r)

        b0 = 2 * k
        r = half(b0, b0 + 2, rows_a, sem_a, r)
        r = half(b0 + 1, b0 + 3, rows_b, sem_b, r)
        return r

    lax.fori_loop(0, _BPW // 2, step, zeros)
    pltpu.sync_copy(out2_v, out_hbm.at[wid])


def kernel(x, tables):
    z = _transpose_tables(tables)
    xp = jnp.pad(x.astype(jnp.int32), ((0, 0), (0, _XP - _F))).reshape(-1)
    mesh = plsc.VectorSubcoreMesh(core_axis_name="c", subcore_axis_name="s",
                                  num_cores=_NC, num_subcores=_NS)
    out = pl.kernel(
        _body,
        out_type=jax.ShapeDtypeStruct((_NW, _BPW // _L, _L), jnp.float32),
        mesh=mesh,
        compiler_params=pltpu.CompilerParams(needs_layout_passes=False,
                                             use_tc_tiling_on_sc=False),
        scratch_types=[
            pltpu.VMEM((_BPW * _XP,), jnp.int32),
            pltpu.VMEM((_F, _W), jnp.float32),
            pltpu.VMEM((_F, _W), jnp.float32),
            pltpu.VMEM((_BPW // _L, _L), jnp.float32),
            pltpu.SemaphoreType.DMA,
            pltpu.SemaphoreType.DMA,
        ],
    )(z, xp)
    return out.reshape(_B, 1)
